# baseline (speedup 1.0000x reference)
import functools

import jax
import jax.numpy as jnp
from jax import lax
from jax.experimental import pallas as pl
from jax.experimental.pallas import tpu as pltpu
from jax.experimental.pallas import tpu_sc as plsc

_N = 10000
_E = 320000
_NH = 128
_NE = 16
_KAPPA = 0.9
_ITERS = 5

_NC = 2
_NS = 16
_NW = _NC * _NS
_CH = 256
_NCHUNK = _E // _CH
_IPC = _CH // 128
_TROWS = _N // _NS

_HI = lax.Precision.HIGHEST


_NHH = _NH // _NC


def _sc_edge_body(r2_hbm, s2_hbm, h2_hbm, z_hbm, y_hbm, a_hbm,
                  he_hbm, acc128_hbm, acc16_hbm,
                  idx_r, idx_s, hbuf, zr, zs, abuf, hebuf, ybuf,
                  gsem, acc128_sh, acc16_sh):
    c = lax.axis_index("c")
    s = lax.axis_index("s")
    w = s * _NC + c
    base_n = s * _TROWS

    def _zero_body(e, _):
        hebuf[0][e, :] = jnp.zeros((16,), jnp.float32)
        for v in range(_NHH // 16):
            ybuf[0][e, pl.ds(v * 16, 16)] = jnp.zeros((16,), jnp.float32)
        return 0
    lax.fori_loop(0, _CH, _zero_body, 0)
    off = 0
    while off < _TROWS:
        sz = min(_CH, _TROWS - off)
        pltpu.sync_copy(ybuf[0].at[pl.ds(0, sz)],
                        acc128_sh.at[pl.ds(base_n + off, sz)])
        pltpu.sync_copy(hebuf[0].at[pl.ds(0, sz)],
                        acc16_sh.at[pl.ds(base_n + off, sz)])
        off += sz
    plsc.subcore_barrier()

    nchunks1 = (_NCHUNK - w + _NW - 1) // _NW

    def _issue1(b, r):
        chunk = w + r * _NW
        pltpu.sync_copy(r2_hbm.at[chunk], idx_r[b])
        pltpu.sync_copy(s2_hbm.at[chunk], idx_s[b])
        pltpu.sync_copy(a_hbm.at[pl.ds(chunk * _CH, _CH)], abuf[b])
        for j in range(_IPC):
            pltpu.async_copy(z_hbm.at[idx_r[b].at[j]],
                             zr[b].at[pl.ds(j * 128, 128)], gsem[b])
            pltpu.async_copy(z_hbm.at[idx_s[b].at[j]],
                             zs[b].at[pl.ds(j * 128, 128)], gsem[b])

    def _drain1(b):
        for j in range(_IPC):
            pltpu.make_async_copy(z_hbm.at[idx_r[b].at[j]],
                                  zr[b].at[pl.ds(j * 128, 128)],
                                  gsem[b]).wait()
            pltpu.make_async_copy(z_hbm.at[idx_s[b].at[j]],
                                  zs[b].at[pl.ds(j * 128, 128)],
                                  gsem[b]).wait()

    def _compute1(b, r):
        chunk = w + r * _NW
        cbase = chunk * _CH

        def _he(e4, _):
            for k in range(4):
                e = e4 * 4 + k
                row = abuf[b][e, :] + zr[b][e, :] + zs[b][e, :]
                hebuf[b][e, :] = jnp.maximum(row, 0.0)
            return 0
        lax.fori_loop(0, _CH // 4, _he, 0)
        pltpu.sync_copy(hebuf[b], he_hbm.at[pl.ds(cbase, _CH)])
        for j in range(_IPC):
            pltpu.sync_copy(hebuf[b].at[pl.ds(j * 128, 128)],
                            acc16_sh.at[idx_s[b].at[j]], add=True)

    _issue1(0, 0)

    def _outer1(i, _):
        r0 = i * 2
        for b in range(2):
            r = r0 + b

            @pl.when(r < nchunks1)
            def _():
                @pl.when(r + 1 < nchunks1)
                def _():
                    _issue1(1 - b, r + 1)
                _drain1(b)
                _compute1(b, r)
        return 0

    lax.fori_loop(0, (nchunks1 + 1) // 2, _outer1, 0)

    nchunks2 = (_NCHUNK - s + _NS - 1) // _NS
    yh_hbm = y_hbm.at[c]

    def _issue2(b, r):
        chunk = s + r * _NS
        pltpu.sync_copy(r2_hbm.at[chunk], idx_r[b])
        pltpu.sync_copy(s2_hbm.at[chunk], idx_s[b])
        pltpu.sync_copy(h2_hbm.at[chunk], hbuf[b])
        for j in range(_IPC):
            pltpu.async_copy(yh_hbm.at[idx_r[b].at[j]],
                             ybuf[b].at[pl.ds(j * 128, 128)], gsem[b])

    def _drain2(b):
        for j in range(_IPC):
            pltpu.make_async_copy(yh_hbm.at[idx_r[b].at[j]],
                                  ybuf[b].at[pl.ds(j * 128, 128)],
                                  gsem[b]).wait()

    def _compute2(b, r):
        for j in range(_IPC):
            def _scale(g, _, j=j):
                h16 = hbuf[b][j, pl.ds(g * 16, 16)]
                for k in range(16):
                    e = j * 128 + g * 16 + k
                    hv = jnp.full((16,), h16[k], jnp.float32)
                    for v in range(_NHH // 16):
                        sl = pl.ds(v * 16, 16)
                        ybuf[b][e, sl] = ybuf[b][e, sl] * hv
                return 0
            lax.fori_loop(0, 8, _scale, 0)
        for j in range(_IPC):
            pltpu.sync_copy(ybuf[b].at[pl.ds(j * 128, 128)],
                            acc128_sh.at[idx_s[b].at[j]], add=True)

    _issue2(0, 0)

    def _outer2(i, _):
        r0 = i * 2
        for b in range(2):
            r = r0 + b

            @pl.when(r < nchunks2)
            def _():
                @pl.when(r + 1 < nchunks2)
                def _():
                    _issue2(1 - b, r + 1)
                _drain2(b)
                _compute2(b, r)
        return 0

    lax.fori_loop(0, (nchunks2 + 1) // 2, _outer2, 0)

    plsc.subcore_barrier()
    pltpu.sync_copy(acc128_sh.at[pl.ds(base_n, _TROWS)],
                    acc128_hbm.at[c, s])
    pltpu.sync_copy(acc16_sh.at[pl.ds(base_n, _TROWS)],
                    acc16_hbm.at[c, s])


_sc_edge = pl.kernel(
    _sc_edge_body,
    out_type=[
        jax.ShapeDtypeStruct((_E, _NE), jnp.float32),
        jax.ShapeDtypeStruct((_NC, _NS, _TROWS, _NHH), jnp.float32),
        jax.ShapeDtypeStruct((_NC, _NS, _TROWS, _NE), jnp.float32),
    ],
    mesh=plsc.VectorSubcoreMesh(core_axis_name="c", subcore_axis_name="s"),
    compiler_params=pltpu.CompilerParams(use_tc_tiling_on_sc=False),
    scratch_types=[
        [pltpu.VMEM((_IPC, 128), jnp.int32)] * 2,
        [pltpu.VMEM((_IPC, 128), jnp.int32)] * 2,
        [pltpu.VMEM((_IPC, 128), jnp.float32)] * 2,
        [pltpu.VMEM((_CH, _NE), jnp.float32)] * 2,
        [pltpu.VMEM((_CH, _NE), jnp.float32)] * 2,
        [pltpu.VMEM((_CH, _NE), jnp.float32)] * 2,
        [pltpu.VMEM((_CH, _NE), jnp.float32)] * 2,
        [pltpu.VMEM((_CH, _NHH), jnp.float32)] * 2,
        [pltpu.SemaphoreType.DMA] * 2,
        pltpu.VMEM_SHARED((_N, _NHH), jnp.float32),
        pltpu.VMEM_SHARED((_N, _NE), jnp.float32),
    ],
)


def _u_body(nd_ref, om_ref, u_ref):
    u_ref[...] = lax.dot_general(
        nd_ref[...], om_ref[...], (((0,), (1,)), ((), ())),
        precision=_HI, preferred_element_type=jnp.float32)


def _tc_u(node_data, Omega):
    return pl.pallas_call(
        _u_body,
        out_shape=jax.ShapeDtypeStruct((_N, _NH), jnp.float32),
    )(node_data, Omega)


_BE = 6400


def _ue_body(ra_ref, oe_ref, ue_ref):
    ue_ref[...] = lax.dot_general(
        ra_ref[...], oe_ref[...], (((0,), (1,)), ((), ())),
        precision=_HI, preferred_element_type=jnp.float32)


def _tc_ue(Ra_data, Omega_e):
    return pl.pallas_call(
        _ue_body,
        grid=(_E // _BE,),
        in_specs=[
            pl.BlockSpec((_NE, _BE), lambda i: (0, i)),
            pl.BlockSpec((_NE, _NE), lambda i: (0, 0)),
        ],
        out_specs=pl.BlockSpec((_BE, _NE), lambda i: (i, 0)),
        out_shape=jax.ShapeDtypeStruct((_E, _NE), jnp.float32),
    )(Ra_data, Omega_e)


def _ea_body(he_ref, ue_ref, we_ref, a_ref):
    a_ref[...] = _KAPPA * lax.dot_general(
        he_ref[...], we_ref[...], (((1,), (1,)), ((), ())),
        precision=_HI, preferred_element_type=jnp.float32) + ue_ref[...]


def _tc_edgea(He, Ue, W_e):
    return pl.pallas_call(
        _ea_body,
        grid=(_E // _BE,),
        in_specs=[
            pl.BlockSpec((_BE, _NE), lambda i: (i, 0)),
            pl.BlockSpec((_BE, _NE), lambda i: (i, 0)),
            pl.BlockSpec((_NE, _NE), lambda i: (0, 0)),
        ],
        out_specs=pl.BlockSpec((_BE, _NE), lambda i: (i, 0)),
        out_shape=jax.ShapeDtypeStruct((_E, _NE), jnp.float32),
    )(He, Ue, W_e)


_BN = 2000


def _upd_body(a128_ref, a16_ref, u_ref, w_ref, bne_ref, ben_ref,
              x_ref, y_ref, z_ref):
    acc = jnp.concatenate([a128_ref[0], a128_ref[1]], axis=1)
    e2n = a16_ref[0] + a16_ref[1]
    x = jnp.maximum(
        acc + lax.dot_general(e2n, ben_ref[...], (((1,), (1,)), ((), ())),
                              precision=_HI,
                              preferred_element_type=jnp.float32)
        + u_ref[...], 0.0)
    x_ref[...] = x
    for cc in range(_NC):
        wh = w_ref[pl.ds(cc * _NHH, _NHH), :]
        y_ref[cc] = _KAPPA * lax.dot_general(
            x, wh, (((1,), (1,)), ((), ())),
            precision=_HI, preferred_element_type=jnp.float32)
    z_ref[...] = lax.dot_general(
        x, bne_ref[...], (((1,), (1,)), ((), ())),
        precision=_HI, preferred_element_type=jnp.float32)


def _tc_update(a128, a16, U, W, B_ne, B_en):
    return pl.pallas_call(
        _upd_body,
        grid=(_N // _BN,),
        in_specs=[
            pl.BlockSpec((_NC, _BN, _NHH), lambda i: (0, i, 0)),
            pl.BlockSpec((_NC, _BN, _NE), lambda i: (0, i, 0)),
            pl.BlockSpec((_BN, _NH), lambda i: (i, 0)),
            pl.BlockSpec((_NH, _NH), lambda i: (0, 0)),
            pl.BlockSpec((_NE, _NH), lambda i: (0, 0)),
            pl.BlockSpec((_NH, _NE), lambda i: (0, 0)),
        ],
        out_specs=[
            pl.BlockSpec((_BN, _NH), lambda i: (i, 0)),
            pl.BlockSpec((_NC, _BN, _NHH), lambda i: (0, i, 0)),
            pl.BlockSpec((_BN, _NE), lambda i: (i, 0)),
        ],
        out_shape=[
            jax.ShapeDtypeStruct((_N, _NH), jnp.float32),
            jax.ShapeDtypeStruct((_NC, _N, _NHH), jnp.float32),
            jax.ShapeDtypeStruct((_N, _NE), jnp.float32),
        ],
    )(a128, a16, U, W, B_ne, B_en)


def _ro_body(x_ref, v0w_ref, v0b_ref, v1w_ref, v1b_ref, o_ref):
    hdd = jnp.maximum(
        lax.dot_general(x_ref[...], v0w_ref[...], (((1,), (1,)), ((), ())),
                        precision=_HI, preferred_element_type=jnp.float32)
        + v0b_ref[...][None, :], 0.0)
    o_ref[...] = lax.dot_general(
        hdd, v1w_ref[...], (((1,), (1,)), ((), ())),
        precision=_HI, preferred_element_type=jnp.float32) \
        + v1b_ref[...][None, :]


def _tc_readout(X, V0_w, V0_b, V1_w, V1_b):
    return pl.pallas_call(
        _ro_body,
        grid=(_N // _BN,),
        in_specs=[
            pl.BlockSpec((_BN, _NH), lambda i: (i, 0)),
            pl.BlockSpec((_NH, _NH), lambda i: (0, 0)),
            pl.BlockSpec((_NH,), lambda i: (0,)),
            pl.BlockSpec((_NH, _NH), lambda i: (0, 0)),
            pl.BlockSpec((_NH,), lambda i: (0,)),
        ],
        out_specs=pl.BlockSpec((_BN, _NH), lambda i: (i, 0)),
        out_shape=jax.ShapeDtypeStruct((_N, _NH), jnp.float32),
    )(X, V0_w, V0_b, V1_w, V1_b)


def _lg_body(he_ref, p3_ref, o_ref):
    o_ref[...] = lax.dot_general(
        he_ref[...], p3_ref[...], (((1,), (1,)), ((), ())),
        precision=_HI, preferred_element_type=jnp.float32)


def _tc_logits(He, P3):
    return pl.pallas_call(
        _lg_body,
        grid=(_E // _BE,),
        in_specs=[
            pl.BlockSpec((_BE, _NE), lambda i: (i, 0)),
            pl.BlockSpec((3, _NE), lambda i: (0, 0)),
        ],
        out_specs=pl.BlockSpec((_BE, 3), lambda i: (i, 0)),
        out_shape=jax.ShapeDtypeStruct((_E, 3), jnp.float32),
    )(He, P3)


def _ht_body(he_ref, o_ref):
    o_ref[...] = he_ref[...].T


def _tc_het(He):
    return pl.pallas_call(
        _ht_body,
        grid=(_E // _BE,),
        in_specs=[pl.BlockSpec((_BE, _NE), lambda i: (i, 0))],
        out_specs=pl.BlockSpec((_NE, _BE), lambda i: (0, i)),
        out_shape=jax.ShapeDtypeStruct((_NE, _E), jnp.float32),
    )(He)


def kernel(R, S, H, node_data, Ra_data, W, Omega, W_e, Omega_e,
           B_ne, B_en, P3, V0_w, V0_b, V1_w, V1_b):
    r2 = R.reshape(_NCHUNK, _IPC, 128)
    s2 = S.reshape(_NCHUNK, _IPC, 128)
    h2 = H.reshape(_NCHUNK, _IPC, 128)

    U = _tc_u(node_data, Omega)
    Ue = _tc_ue(Ra_data, Omega_e)

    He = jnp.zeros((_E, _NE), jnp.float32)
    Y = jnp.zeros((_NC, _N, _NHH), jnp.float32)
    Z = jnp.zeros((_N, _NE), jnp.float32)
    X = jnp.zeros((_N, _NH), jnp.float32)
    for _ in range(_ITERS):
        A = _tc_edgea(He, Ue, W_e)
        He, a128, a16 = _sc_edge(r2, s2, h2, Z, Y, A)
        X, Y, Z = _tc_update(a128.reshape(_NC, _N, _NHH),
                             a16.reshape(_NC, _N, _NE), U, W, B_ne, B_en)

    x = _tc_readout(X, V0_w, V0_b, V1_w, V1_b)
    logits = _tc_logits(He, P3)
    He_T = _tc_het(He)
    return (x, He_T, logits)


# speedup vs baseline: 1.3068x; 1.3068x over previous
import functools

import jax
import jax.numpy as jnp
from jax import lax
from jax.experimental import pallas as pl
from jax.experimental.pallas import tpu as pltpu
from jax.experimental.pallas import tpu_sc as plsc

_N = 10000
_E = 320000
_NH = 128
_NE = 16
_KAPPA = 0.9
_ITERS = 5

_NC = 2
_NS = 16
_NW = _NC * _NS
_CH = 512
_NCHUNK = _E // _CH
_IPC = _CH // 128
_TROWS = _N // _NS

_HI = lax.Precision.HIGHEST


_NHH = _NH // _NC


def _sc_edge_body(r2_hbm, s2_hbm, h2_hbm, z_hbm, y_hbm, a_hbm,
                  he_hbm, acc128_hbm, acc16_hbm,
                  idx_r, idx_s, hbuf, zr, zs, abuf, hebuf, ybuf,
                  gsem, acc128_sh, acc16_sh):
    c = lax.axis_index("c")
    s = lax.axis_index("s")
    w = s * _NC + c
    base_n = s * _TROWS

    def _zero_body(e, _):
        hebuf[0][e, :] = jnp.zeros((16,), jnp.float32)
        for v in range(_NHH // 16):
            ybuf[0][e, pl.ds(v * 16, 16)] = jnp.zeros((16,), jnp.float32)
        return 0
    lax.fori_loop(0, _CH, _zero_body, 0)
    off = 0
    while off < _TROWS:
        sz = min(_CH, _TROWS - off)
        pltpu.sync_copy(ybuf[0].at[pl.ds(0, sz)],
                        acc128_sh.at[pl.ds(base_n + off, sz)])
        pltpu.sync_copy(hebuf[0].at[pl.ds(0, sz)],
                        acc16_sh.at[pl.ds(base_n + off, sz)])
        off += sz
    plsc.subcore_barrier()

    nchunks1 = (_NCHUNK - w + _NW - 1) // _NW

    def _issue1(b, r):
        chunk = w + r * _NW
        pltpu.sync_copy(r2_hbm.at[chunk], idx_r[b])
        pltpu.sync_copy(s2_hbm.at[chunk], idx_s[b])
        pltpu.sync_copy(a_hbm.at[pl.ds(chunk * _CH, _CH)], abuf[b])
        for j in range(_IPC):
            pltpu.async_copy(z_hbm.at[idx_r[b].at[j]],
                             zr[b].at[pl.ds(j * 128, 128)], gsem[b])
            pltpu.async_copy(z_hbm.at[idx_s[b].at[j]],
                             zs[b].at[pl.ds(j * 128, 128)], gsem[b])

    def _drain1(b):
        for j in range(_IPC):
            pltpu.make_async_copy(z_hbm.at[idx_r[b].at[j]],
                                  zr[b].at[pl.ds(j * 128, 128)],
                                  gsem[b]).wait()
            pltpu.make_async_copy(z_hbm.at[idx_s[b].at[j]],
                                  zs[b].at[pl.ds(j * 128, 128)],
                                  gsem[b]).wait()

    def _compute1(b, r):
        chunk = w + r * _NW
        cbase = chunk * _CH

        def _he(e4, _):
            for k in range(4):
                e = e4 * 4 + k
                row = abuf[b][e, :] + zr[b][e, :] + zs[b][e, :]
                hebuf[b][e, :] = jnp.maximum(row, 0.0)
            return 0
        lax.fori_loop(0, _CH // 4, _he, 0)
        pltpu.sync_copy(hebuf[b], he_hbm.at[pl.ds(cbase, _CH)])
        for j in range(_IPC):
            pltpu.sync_copy(hebuf[b].at[pl.ds(j * 128, 128)],
                            acc16_sh.at[idx_s[b].at[j]], add=True)

    def _round1(r, _):
        _issue1(0, r)
        _drain1(0)
        _compute1(0, r)
        return 0

    lax.fori_loop(0, nchunks1, _round1, 0)

    nchunks2 = (_NCHUNK - s + _NS - 1) // _NS
    yh_hbm = y_hbm.at[c]

    def _issue2(b, r):
        chunk = s + r * _NS
        pltpu.sync_copy(r2_hbm.at[chunk], idx_r[b])
        pltpu.sync_copy(s2_hbm.at[chunk], idx_s[b])
        pltpu.sync_copy(h2_hbm.at[chunk], hbuf[b])
        for j in range(_IPC):
            pltpu.async_copy(yh_hbm.at[idx_r[b].at[j]],
                             ybuf[b].at[pl.ds(j * 128, 128)], gsem[b])

    def _drain2(b):
        for j in range(_IPC):
            pltpu.make_async_copy(yh_hbm.at[idx_r[b].at[j]],
                                  ybuf[b].at[pl.ds(j * 128, 128)],
                                  gsem[b]).wait()

    def _compute2(b, r):
        for j in range(_IPC):
            def _scale(g, _, j=j):
                h16 = hbuf[b][j, pl.ds(g * 16, 16)]
                for k in range(16):
                    e = j * 128 + g * 16 + k
                    hv = jnp.full((16,), h16[k], jnp.float32)
                    for v in range(_NHH // 16):
                        sl = pl.ds(v * 16, 16)
                        ybuf[b][e, sl] = ybuf[b][e, sl] * hv
                return 0
            lax.fori_loop(0, 8, _scale, 0)
        for j in range(_IPC):
            pltpu.sync_copy(ybuf[b].at[pl.ds(j * 128, 128)],
                            acc128_sh.at[idx_s[b].at[j]], add=True)

    def _round2(r, _):
        _issue2(0, r)
        _drain2(0)
        _compute2(0, r)
        return 0

    lax.fori_loop(0, nchunks2, _round2, 0)

    plsc.subcore_barrier()
    pltpu.sync_copy(acc128_sh.at[pl.ds(base_n, _TROWS)],
                    acc128_hbm.at[c, s])
    pltpu.sync_copy(acc16_sh.at[pl.ds(base_n, _TROWS)],
                    acc16_hbm.at[c, s])


_sc_edge = pl.kernel(
    _sc_edge_body,
    out_type=[
        jax.ShapeDtypeStruct((_E, _NE), jnp.float32),
        jax.ShapeDtypeStruct((_NC, _NS, _TROWS, _NHH), jnp.float32),
        jax.ShapeDtypeStruct((_NC, _NS, _TROWS, _NE), jnp.float32),
    ],
    mesh=plsc.VectorSubcoreMesh(core_axis_name="c", subcore_axis_name="s"),
    compiler_params=pltpu.CompilerParams(use_tc_tiling_on_sc=False),
    scratch_types=[
        [pltpu.VMEM((_IPC, 128), jnp.int32)],
        [pltpu.VMEM((_IPC, 128), jnp.int32)],
        [pltpu.VMEM((_IPC, 128), jnp.float32)],
        [pltpu.VMEM((_CH, _NE), jnp.float32)],
        [pltpu.VMEM((_CH, _NE), jnp.float32)],
        [pltpu.VMEM((_CH, _NE), jnp.float32)],
        [pltpu.VMEM((_CH, _NE), jnp.float32)],
        [pltpu.VMEM((_CH, _NHH), jnp.float32)],
        [pltpu.SemaphoreType.DMA],
        pltpu.VMEM_SHARED((_N, _NHH), jnp.float32),
        pltpu.VMEM_SHARED((_N, _NE), jnp.float32),
    ],
)


def _sc_s16_body(s2_hbm, he_hbm, acc16_out, idx_s, hebuf, gsem, acc16_sh):
    c = lax.axis_index("c")
    s = lax.axis_index("s")
    w = s * _NC + c
    base_n = s * _TROWS

    def _zero_body(e, _):
        hebuf[e, :] = jnp.zeros((16,), jnp.float32)
        return 0
    lax.fori_loop(0, _CH, _zero_body, 0)
    off = 0
    while off < _TROWS:
        sz = min(_CH, _TROWS - off)
        pltpu.sync_copy(hebuf.at[pl.ds(0, sz)],
                        acc16_sh.at[pl.ds(base_n + off, sz)])
        off += sz
    plsc.subcore_barrier()

    nchunks = (_NCHUNK - w + _NW - 1) // _NW

    def _round(r, _):
        chunk = w + r * _NW
        pltpu.sync_copy(s2_hbm.at[chunk], idx_s)
        pltpu.sync_copy(he_hbm.at[pl.ds(chunk * _CH, _CH)], hebuf)
        for j in range(_IPC):
            pltpu.sync_copy(hebuf.at[pl.ds(j * 128, 128)],
                            acc16_sh.at[idx_s.at[j]], add=True)
        return 0

    lax.fori_loop(0, nchunks, _round, 0)
    plsc.subcore_barrier()
    pltpu.sync_copy(acc16_sh.at[pl.ds(base_n, _TROWS)],
                    acc16_out.at[c, s])


_sc_scatter16 = pl.kernel(
    _sc_s16_body,
    out_type=jax.ShapeDtypeStruct((_NC, _NS, _TROWS, _NE), jnp.float32),
    mesh=plsc.VectorSubcoreMesh(core_axis_name="c", subcore_axis_name="s"),
    compiler_params=pltpu.CompilerParams(use_tc_tiling_on_sc=False),
    scratch_types=[
        pltpu.VMEM((_IPC, 128), jnp.int32),
        pltpu.VMEM((_CH, _NE), jnp.float32),
        pltpu.SemaphoreType.DMA,
        pltpu.VMEM_SHARED((_N, _NE), jnp.float32),
    ],
)


def _u_body(nd_ref, om_ref, u_ref):
    u_ref[...] = lax.dot_general(
        nd_ref[...], om_ref[...], (((0,), (1,)), ((), ())),
        precision=_HI, preferred_element_type=jnp.float32)


def _tc_u(node_data, Omega):
    return pl.pallas_call(
        _u_body,
        out_shape=jax.ShapeDtypeStruct((_N, _NH), jnp.float32),
    )(node_data, Omega)


_BE = 6400


def _ue_body(ra_ref, oe_ref, ue_ref, he1_ref):
    ue = lax.dot_general(
        ra_ref[...], oe_ref[...], (((0,), (1,)), ((), ())),
        precision=_HI, preferred_element_type=jnp.float32)
    ue_ref[...] = ue
    he1_ref[...] = jnp.maximum(ue, 0.0)


def _tc_ue(Ra_data, Omega_e):
    return pl.pallas_call(
        _ue_body,
        grid=(_E // _BE,),
        in_specs=[
            pl.BlockSpec((_NE, _BE), lambda i: (0, i)),
            pl.BlockSpec((_NE, _NE), lambda i: (0, 0)),
        ],
        out_specs=[
            pl.BlockSpec((_BE, _NE), lambda i: (i, 0)),
            pl.BlockSpec((_BE, _NE), lambda i: (i, 0)),
        ],
        out_shape=[
            jax.ShapeDtypeStruct((_E, _NE), jnp.float32),
            jax.ShapeDtypeStruct((_E, _NE), jnp.float32),
        ],
    )(Ra_data, Omega_e)


def _ea_body(he_ref, ue_ref, we_ref, a_ref):
    a_ref[...] = _KAPPA * lax.dot_general(
        he_ref[...], we_ref[...], (((1,), (1,)), ((), ())),
        precision=_HI, preferred_element_type=jnp.float32) + ue_ref[...]


def _tc_edgea(He, Ue, W_e):
    return pl.pallas_call(
        _ea_body,
        grid=(_E // _BE,),
        in_specs=[
            pl.BlockSpec((_BE, _NE), lambda i: (i, 0)),
            pl.BlockSpec((_BE, _NE), lambda i: (i, 0)),
            pl.BlockSpec((_NE, _NE), lambda i: (0, 0)),
        ],
        out_specs=pl.BlockSpec((_BE, _NE), lambda i: (i, 0)),
        out_shape=jax.ShapeDtypeStruct((_E, _NE), jnp.float32),
    )(He, Ue, W_e)


_BN = 2000


def _upd_body(a128_ref, a16_ref, u_ref, w_ref, bne_ref, ben_ref,
              x_ref, y_ref, z_ref):
    acc = jnp.concatenate([a128_ref[0], a128_ref[1]], axis=1)
    e2n = a16_ref[0] + a16_ref[1]
    x = jnp.maximum(
        acc + lax.dot_general(e2n, ben_ref[...], (((1,), (1,)), ((), ())),
                              precision=_HI,
                              preferred_element_type=jnp.float32)
        + u_ref[...], 0.0)
    x_ref[...] = x
    for cc in range(_NC):
        wh = w_ref[pl.ds(cc * _NHH, _NHH), :]
        y_ref[cc] = _KAPPA * lax.dot_general(
            x, wh, (((1,), (1,)), ((), ())),
            precision=_HI, preferred_element_type=jnp.float32)
    z_ref[...] = lax.dot_general(
        x, bne_ref[...], (((1,), (1,)), ((), ())),
        precision=_HI, preferred_element_type=jnp.float32)


def _tc_update(a128, a16, U, W, B_ne, B_en):
    return pl.pallas_call(
        _upd_body,
        grid=(_N // _BN,),
        in_specs=[
            pl.BlockSpec((_NC, _BN, _NHH), lambda i: (0, i, 0)),
            pl.BlockSpec((_NC, _BN, _NE), lambda i: (0, i, 0)),
            pl.BlockSpec((_BN, _NH), lambda i: (i, 0)),
            pl.BlockSpec((_NH, _NH), lambda i: (0, 0)),
            pl.BlockSpec((_NE, _NH), lambda i: (0, 0)),
            pl.BlockSpec((_NH, _NE), lambda i: (0, 0)),
        ],
        out_specs=[
            pl.BlockSpec((_BN, _NH), lambda i: (i, 0)),
            pl.BlockSpec((_NC, _BN, _NHH), lambda i: (0, i, 0)),
            pl.BlockSpec((_BN, _NE), lambda i: (i, 0)),
        ],
        out_shape=[
            jax.ShapeDtypeStruct((_N, _NH), jnp.float32),
            jax.ShapeDtypeStruct((_NC, _N, _NHH), jnp.float32),
            jax.ShapeDtypeStruct((_N, _NE), jnp.float32),
        ],
    )(a128, a16, U, W, B_ne, B_en)


def _ro_body(x_ref, v0w_ref, v0b_ref, v1w_ref, v1b_ref, o_ref):
    hdd = jnp.maximum(
        lax.dot_general(x_ref[...], v0w_ref[...], (((1,), (1,)), ((), ())),
                        precision=_HI, preferred_element_type=jnp.float32)
        + v0b_ref[...][None, :], 0.0)
    o_ref[...] = lax.dot_general(
        hdd, v1w_ref[...], (((1,), (1,)), ((), ())),
        precision=_HI, preferred_element_type=jnp.float32) \
        + v1b_ref[...][None, :]


def _tc_readout(X, V0_w, V0_b, V1_w, V1_b):
    return pl.pallas_call(
        _ro_body,
        grid=(_N // _BN,),
        in_specs=[
            pl.BlockSpec((_BN, _NH), lambda i: (i, 0)),
            pl.BlockSpec((_NH, _NH), lambda i: (0, 0)),
            pl.BlockSpec((_NH,), lambda i: (0,)),
            pl.BlockSpec((_NH, _NH), lambda i: (0, 0)),
            pl.BlockSpec((_NH,), lambda i: (0,)),
        ],
        out_specs=pl.BlockSpec((_BN, _NH), lambda i: (i, 0)),
        out_shape=jax.ShapeDtypeStruct((_N, _NH), jnp.float32),
    )(X, V0_w, V0_b, V1_w, V1_b)


def _lg_body(he_ref, p3_ref, o_ref):
    o_ref[...] = lax.dot_general(
        he_ref[...], p3_ref[...], (((1,), (1,)), ((), ())),
        precision=_HI, preferred_element_type=jnp.float32)


def _tc_logits(He, P3):
    return pl.pallas_call(
        _lg_body,
        grid=(_E // _BE,),
        in_specs=[
            pl.BlockSpec((_BE, _NE), lambda i: (i, 0)),
            pl.BlockSpec((3, _NE), lambda i: (0, 0)),
        ],
        out_specs=pl.BlockSpec((_BE, 3), lambda i: (i, 0)),
        out_shape=jax.ShapeDtypeStruct((_E, 3), jnp.float32),
    )(He, P3)


def _ht_body(he_ref, o_ref):
    o_ref[...] = he_ref[...].T


def _tc_het(He):
    return pl.pallas_call(
        _ht_body,
        grid=(_E // _BE,),
        in_specs=[pl.BlockSpec((_BE, _NE), lambda i: (i, 0))],
        out_specs=pl.BlockSpec((_NE, _BE), lambda i: (0, i)),
        out_shape=jax.ShapeDtypeStruct((_NE, _E), jnp.float32),
    )(He)


def kernel(R, S, H, node_data, Ra_data, W, Omega, W_e, Omega_e,
           B_ne, B_en, P3, V0_w, V0_b, V1_w, V1_b):
    r2 = R.reshape(_NCHUNK, _IPC, 128)
    s2 = S.reshape(_NCHUNK, _IPC, 128)
    h2 = H.reshape(_NCHUNK, _IPC, 128)

    U = _tc_u(node_data, Omega)
    Ue, He = _tc_ue(Ra_data, Omega_e)

    a16 = _sc_scatter16(s2, He)
    X, Y, Z = _tc_update(jnp.zeros((_NC, _N, _NHH), jnp.float32),
                         a16.reshape(_NC, _N, _NE), U, W, B_ne, B_en)
    for _ in range(_ITERS - 1):
        A = _tc_edgea(He, Ue, W_e)
        He, a128, a16 = _sc_edge(r2, s2, h2, Z, Y, A)
        X, Y, Z = _tc_update(a128.reshape(_NC, _N, _NHH),
                             a16.reshape(_NC, _N, _NE), U, W, B_ne, B_en)

    x = _tc_readout(X, V0_w, V0_b, V1_w, V1_b)
    logits = _tc_logits(He, P3)
    He_T = _tc_het(He)
    return (x, He_T, logits)


# speedup vs baseline: 1.7749x; 1.3582x over previous
import functools

import jax
import jax.numpy as jnp
from jax import lax
from jax.experimental import pallas as pl
from jax.experimental.pallas import tpu as pltpu
from jax.experimental.pallas import tpu_sc as plsc

_N = 10000
_E = 320000
_NH = 128
_NE = 16
_KAPPA = 0.9
_ITERS = 5

_NC = 2
_NS = 16
_NW = _NC * _NS
_CH = 512
_NCHUNK = _E // _CH
_IPC = _CH // 128
_TROWS = _N // _NS

_HI = lax.Precision.HIGHEST


_NHH = _NH // _NC


def _sc_edge_body(r2_hbm, s2_hbm, h2_hbm, z_hbm, y_hbm, a_hbm,
                  he_hbm, acc128_hbm, acc16_hbm,
                  idx_r, idx_s, hbuf, zr, zs, abuf, hebuf, ybuf,
                  gsem, acc128_sh, acc16_sh):
    c = lax.axis_index("c")
    s = lax.axis_index("s")
    w = s * _NC + c
    base_n = s * _TROWS

    def _zero_body(e, _):
        hebuf[0][e, :] = jnp.zeros((16,), jnp.float32)
        for v in range(_NHH // 16):
            ybuf[0][e, pl.ds(v * 16, 16)] = jnp.zeros((16,), jnp.float32)
        return 0
    lax.fori_loop(0, _CH, _zero_body, 0)
    off = 0
    while off < _TROWS:
        sz = min(_CH, _TROWS - off)
        pltpu.sync_copy(ybuf[0].at[pl.ds(0, sz)],
                        acc128_sh.at[pl.ds(base_n + off, sz)])
        pltpu.sync_copy(hebuf[0].at[pl.ds(0, sz)],
                        acc16_sh.at[pl.ds(base_n + off, sz)])
        off += sz
    plsc.subcore_barrier()

    nchunks1 = (_NCHUNK - w + _NW - 1) // _NW

    def _issue1(b, r):
        chunk = w + r * _NW
        pltpu.sync_copy(r2_hbm.at[chunk], idx_r[b])
        pltpu.sync_copy(s2_hbm.at[chunk], idx_s[b])
        pltpu.sync_copy(a_hbm.at[pl.ds(chunk * _CH, _CH)], abuf[b])
        for j in range(_IPC):
            pltpu.async_copy(z_hbm.at[idx_r[b].at[j]],
                             zr[b].at[pl.ds(j * 128, 128)], gsem[b])
            pltpu.async_copy(z_hbm.at[idx_s[b].at[j]],
                             zs[b].at[pl.ds(j * 128, 128)], gsem[b])

    def _drain1(b):
        for j in range(_IPC):
            pltpu.make_async_copy(z_hbm.at[idx_r[b].at[j]],
                                  zr[b].at[pl.ds(j * 128, 128)],
                                  gsem[b]).wait()
            pltpu.make_async_copy(z_hbm.at[idx_s[b].at[j]],
                                  zs[b].at[pl.ds(j * 128, 128)],
                                  gsem[b]).wait()

    def _compute1(b, r):
        chunk = w + r * _NW
        cbase = chunk * _CH

        def _he(e4, _):
            for k in range(4):
                e = e4 * 4 + k
                row = abuf[b][e, :] + zr[b][e, :] + zs[b][e, :]
                hebuf[b][e, :] = jnp.maximum(row, 0.0)
            return 0
        lax.fori_loop(0, _CH // 4, _he, 0)
        pltpu.sync_copy(hebuf[b], he_hbm.at[pl.ds(cbase, _CH)])
        for j in range(_IPC):
            pltpu.sync_copy(hebuf[b].at[pl.ds(j * 128, 128)],
                            acc16_sh.at[idx_s[b].at[j]], add=True)

    def _round1(r, _):
        _issue1(0, r)
        _drain1(0)
        _compute1(0, r)
        return 0

    lax.fori_loop(0, nchunks1, _round1, 0)

    nchunks2 = (_NCHUNK - s + _NS - 1) // _NS
    yh_hbm = y_hbm.at[c]

    def _issue2(b, r):
        chunk = s + r * _NS
        pltpu.sync_copy(r2_hbm.at[chunk], idx_r[b])
        pltpu.sync_copy(s2_hbm.at[chunk], idx_s[b])
        pltpu.sync_copy(h2_hbm.at[chunk], hbuf[b])
        for j in range(_IPC):
            pltpu.async_copy(yh_hbm.at[idx_r[b].at[j]],
                             ybuf[b].at[pl.ds(j * 128, 128)], gsem[b])

    def _drain2(b):
        for j in range(_IPC):
            pltpu.make_async_copy(yh_hbm.at[idx_r[b].at[j]],
                                  ybuf[b].at[pl.ds(j * 128, 128)],
                                  gsem[b]).wait()

    def _compute2(b, r):
        for j in range(_IPC):
            def _scale(g, _, j=j):
                h16 = hbuf[b][j, pl.ds(g * 16, 16)]
                for k in range(16):
                    e = j * 128 + g * 16 + k
                    hv = jnp.full((16,), h16[k], jnp.float32)
                    for v in range(_NHH // 16):
                        sl = pl.ds(v * 16, 16)
                        ybuf[b][e, sl] = ybuf[b][e, sl] * hv
                return 0
            lax.fori_loop(0, 8, _scale, 0)
        for j in range(_IPC):
            pltpu.sync_copy(ybuf[b].at[pl.ds(j * 128, 128)],
                            acc128_sh.at[idx_s[b].at[j]], add=True)

    def _round2(r, _):
        _issue2(0, r)
        _drain2(0)
        _compute2(0, r)
        return 0

    lax.fori_loop(0, nchunks2, _round2, 0)

    plsc.subcore_barrier()
    pltpu.sync_copy(acc128_sh.at[pl.ds(base_n, _TROWS)],
                    acc128_hbm.at[c, s])
    pltpu.sync_copy(acc16_sh.at[pl.ds(base_n, _TROWS)],
                    acc16_hbm.at[c, s])


_sc_edge = pl.kernel(
    _sc_edge_body,
    out_type=[
        jax.ShapeDtypeStruct((_E, _NE), jnp.float32),
        jax.ShapeDtypeStruct((_NC, _NS, _TROWS, _NHH), jnp.float32),
        jax.ShapeDtypeStruct((_NC, _NS, _TROWS, _NE), jnp.float32),
    ],
    mesh=plsc.VectorSubcoreMesh(core_axis_name="c", subcore_axis_name="s"),
    compiler_params=pltpu.CompilerParams(use_tc_tiling_on_sc=False),
    scratch_types=[
        [pltpu.VMEM((_IPC, 128), jnp.int32)],
        [pltpu.VMEM((_IPC, 128), jnp.int32)],
        [pltpu.VMEM((_IPC, 128), jnp.float32)],
        [pltpu.VMEM((_CH, _NE), jnp.float32)],
        [pltpu.VMEM((_CH, _NE), jnp.float32)],
        [pltpu.VMEM((_CH, _NE), jnp.float32)],
        [pltpu.VMEM((_CH, _NE), jnp.float32)],
        [pltpu.VMEM((_CH, _NHH), jnp.float32)],
        [pltpu.SemaphoreType.DMA],
        pltpu.VMEM_SHARED((_N, _NHH), jnp.float32),
        pltpu.VMEM_SHARED((_N, _NE), jnp.float32),
    ],
)


def _sc_s16_body(s2_hbm, ue_hbm, he_hbm, acc16_out, idx_s, hebuf, gsem,
                 acc16_sh):
    c = lax.axis_index("c")
    s = lax.axis_index("s")
    w = s * _NC + c
    base_n = s * _TROWS

    def _zero_body(e, _):
        hebuf[e, :] = jnp.zeros((16,), jnp.float32)
        return 0
    lax.fori_loop(0, _CH, _zero_body, 0)
    off = 0
    while off < _TROWS:
        sz = min(_CH, _TROWS - off)
        pltpu.sync_copy(hebuf.at[pl.ds(0, sz)],
                        acc16_sh.at[pl.ds(base_n + off, sz)])
        off += sz
    plsc.subcore_barrier()

    nchunks = (_NCHUNK - w + _NW - 1) // _NW

    def _round(r, _):
        chunk = w + r * _NW
        cbase = chunk * _CH
        pltpu.sync_copy(s2_hbm.at[chunk], idx_s)
        pltpu.sync_copy(ue_hbm.at[pl.ds(cbase, _CH)], hebuf)

        def _relu(e4, _):
            for k in range(4):
                e = e4 * 4 + k
                hebuf[e, :] = jnp.maximum(hebuf[e, :], 0.0)
            return 0
        lax.fori_loop(0, _CH // 4, _relu, 0)
        pltpu.sync_copy(hebuf, he_hbm.at[pl.ds(cbase, _CH)])
        for j in range(_IPC):
            pltpu.sync_copy(hebuf.at[pl.ds(j * 128, 128)],
                            acc16_sh.at[idx_s.at[j]], add=True)
        return 0

    lax.fori_loop(0, nchunks, _round, 0)
    plsc.subcore_barrier()
    pltpu.sync_copy(acc16_sh.at[pl.ds(base_n, _TROWS)],
                    acc16_out.at[c, s])


_sc_scatter16 = pl.kernel(
    _sc_s16_body,
    out_type=[
        jax.ShapeDtypeStruct((_E, _NE), jnp.float32),
        jax.ShapeDtypeStruct((_NC, _NS, _TROWS, _NE), jnp.float32),
    ],
    mesh=plsc.VectorSubcoreMesh(core_axis_name="c", subcore_axis_name="s"),
    compiler_params=pltpu.CompilerParams(use_tc_tiling_on_sc=False),
    scratch_types=[
        pltpu.VMEM((_IPC, 128), jnp.int32),
        pltpu.VMEM((_CH, _NE), jnp.float32),
        pltpu.SemaphoreType.DMA,
        pltpu.VMEM_SHARED((_N, _NE), jnp.float32),
    ],
)


def _u_body(nd_ref, om_ref, u_ref):
    u_ref[...] = lax.dot_general(
        nd_ref[...], om_ref[...], (((0,), (1,)), ((), ())),
        precision=_HI, preferred_element_type=jnp.float32)


def _tc_u(node_data, Omega):
    return pl.pallas_call(
        _u_body,
        out_shape=jax.ShapeDtypeStruct((_N, _NH), jnp.float32),
    )(node_data, Omega)


_BE = 6400


def _ue_body(ra_ref, oe_ref, ue_ref):
    ue_ref[...] = lax.dot_general(
        ra_ref[...], oe_ref[...], (((0,), (1,)), ((), ())),
        precision=_HI, preferred_element_type=jnp.float32)


def _tc_ue(Ra_data, Omega_e):
    return pl.pallas_call(
        _ue_body,
        grid=(_E // _BE,),
        in_specs=[
            pl.BlockSpec((_NE, _BE), lambda i: (0, i)),
            pl.BlockSpec((_NE, _NE), lambda i: (0, 0)),
        ],
        out_specs=pl.BlockSpec((_BE, _NE), lambda i: (i, 0)),
        out_shape=jax.ShapeDtypeStruct((_E, _NE), jnp.float32),
    )(Ra_data, Omega_e)


_EF = _E // 8
_BF = 5000


def _ea_body(he_ref, ue_ref, bd_ref, a_ref):
    a_ref[...] = _KAPPA * lax.dot_general(
        he_ref[...], bd_ref[...], (((1,), (0,)), ((), ())),
        precision=_HI, preferred_element_type=jnp.float32) + ue_ref[...]


def _tc_edgea(He_flat, Ue_flat, BD_We):
    return pl.pallas_call(
        _ea_body,
        grid=(_EF // _BF,),
        in_specs=[
            pl.BlockSpec((_BF, 128), lambda i: (i, 0)),
            pl.BlockSpec((_BF, 128), lambda i: (i, 0)),
            pl.BlockSpec((128, 128), lambda i: (0, 0)),
        ],
        out_specs=pl.BlockSpec((_BF, 128), lambda i: (i, 0)),
        out_shape=jax.ShapeDtypeStruct((_EF, 128), jnp.float32),
    )(He_flat, Ue_flat, BD_We)


_BN = 2000


def _upd_body(a128_ref, a16_ref, u_ref, w_ref, bne_ref, ben_ref,
              x_ref, y_ref, z_ref):
    acc = jnp.concatenate([a128_ref[0], a128_ref[1]], axis=1)
    e2n = a16_ref[0] + a16_ref[1]
    x = jnp.maximum(
        acc + lax.dot_general(e2n, ben_ref[...], (((1,), (1,)), ((), ())),
                              precision=_HI,
                              preferred_element_type=jnp.float32)
        + u_ref[...], 0.0)
    x_ref[...] = x
    for cc in range(_NC):
        wh = w_ref[pl.ds(cc * _NHH, _NHH), :]
        y_ref[cc] = _KAPPA * lax.dot_general(
            x, wh, (((1,), (1,)), ((), ())),
            precision=_HI, preferred_element_type=jnp.float32)
    z_ref[...] = lax.dot_general(
        x, bne_ref[...], (((1,), (1,)), ((), ())),
        precision=_HI, preferred_element_type=jnp.float32)


def _tc_update(a128, a16, U, W, B_ne, B_en):
    return pl.pallas_call(
        _upd_body,
        grid=(_N // _BN,),
        in_specs=[
            pl.BlockSpec((_NC, _BN, _NHH), lambda i: (0, i, 0)),
            pl.BlockSpec((_NC, _BN, _NE), lambda i: (0, i, 0)),
            pl.BlockSpec((_BN, _NH), lambda i: (i, 0)),
            pl.BlockSpec((_NH, _NH), lambda i: (0, 0)),
            pl.BlockSpec((_NE, _NH), lambda i: (0, 0)),
            pl.BlockSpec((_NH, _NE), lambda i: (0, 0)),
        ],
        out_specs=[
            pl.BlockSpec((_BN, _NH), lambda i: (i, 0)),
            pl.BlockSpec((_NC, _BN, _NHH), lambda i: (0, i, 0)),
            pl.BlockSpec((_BN, _NE), lambda i: (i, 0)),
        ],
        out_shape=[
            jax.ShapeDtypeStruct((_N, _NH), jnp.float32),
            jax.ShapeDtypeStruct((_NC, _N, _NHH), jnp.float32),
            jax.ShapeDtypeStruct((_N, _NE), jnp.float32),
        ],
    )(a128, a16, U, W, B_ne, B_en)


def _ro_body(x_ref, v0w_ref, v0b_ref, v1w_ref, v1b_ref, o_ref):
    hdd = jnp.maximum(
        lax.dot_general(x_ref[...], v0w_ref[...], (((1,), (1,)), ((), ())),
                        precision=_HI, preferred_element_type=jnp.float32)
        + v0b_ref[...][None, :], 0.0)
    o_ref[...] = lax.dot_general(
        hdd, v1w_ref[...], (((1,), (1,)), ((), ())),
        precision=_HI, preferred_element_type=jnp.float32) \
        + v1b_ref[...][None, :]


def _tc_readout(X, V0_w, V0_b, V1_w, V1_b):
    return pl.pallas_call(
        _ro_body,
        grid=(_N // _BN,),
        in_specs=[
            pl.BlockSpec((_BN, _NH), lambda i: (i, 0)),
            pl.BlockSpec((_NH, _NH), lambda i: (0, 0)),
            pl.BlockSpec((_NH,), lambda i: (0,)),
            pl.BlockSpec((_NH, _NH), lambda i: (0, 0)),
            pl.BlockSpec((_NH,), lambda i: (0,)),
        ],
        out_specs=pl.BlockSpec((_BN, _NH), lambda i: (i, 0)),
        out_shape=jax.ShapeDtypeStruct((_N, _NH), jnp.float32),
    )(X, V0_w, V0_b, V1_w, V1_b)


def _lg_body(he_ref, p3_ref, o_ref):
    o_ref[...] = lax.dot_general(
        he_ref[...], p3_ref[...], (((1,), (1,)), ((), ())),
        precision=_HI, preferred_element_type=jnp.float32)


def _tc_logits(He, P3):
    return pl.pallas_call(
        _lg_body,
        grid=(_E // _BE,),
        in_specs=[
            pl.BlockSpec((_BE, _NE), lambda i: (i, 0)),
            pl.BlockSpec((3, _NE), lambda i: (0, 0)),
        ],
        out_specs=pl.BlockSpec((_BE, 3), lambda i: (i, 0)),
        out_shape=jax.ShapeDtypeStruct((_E, 3), jnp.float32),
    )(He, P3)


def _ht_body(he_ref, o_ref):
    o_ref[...] = he_ref[...].T


def _tc_het(He):
    return pl.pallas_call(
        _ht_body,
        grid=(_E // _BE,),
        in_specs=[pl.BlockSpec((_BE, _NE), lambda i: (i, 0))],
        out_specs=pl.BlockSpec((_NE, _BE), lambda i: (0, i)),
        out_shape=jax.ShapeDtypeStruct((_NE, _E), jnp.float32),
    )(He)


def kernel(R, S, H, node_data, Ra_data, W, Omega, W_e, Omega_e,
           B_ne, B_en, P3, V0_w, V0_b, V1_w, V1_b):
    r2 = R.reshape(_NCHUNK, _IPC, 128)
    s2 = S.reshape(_NCHUNK, _IPC, 128)
    h2 = H.reshape(_NCHUNK, _IPC, 128)

    U = _tc_u(node_data, Omega)
    Ue_flat = _tc_ue(Ra_data, Omega_e).reshape(_EF, 128)
    BD_We = jnp.kron(jnp.eye(8, dtype=jnp.float32), W_e.T)

    He, a16 = _sc_scatter16(s2, Ue_flat.reshape(_E, _NE))
    X, Y, Z = _tc_update(jnp.zeros((_NC, _N, _NHH), jnp.float32),
                         a16.reshape(_NC, _N, _NE), U, W, B_ne, B_en)
    for _ in range(_ITERS - 1):
        A = _tc_edgea(He.reshape(_EF, 128), Ue_flat, BD_We)
        He, a128, a16 = _sc_edge(r2, s2, h2, Z, Y, A.reshape(_E, _NE))
        X, Y, Z = _tc_update(a128.reshape(_NC, _N, _NHH),
                             a16.reshape(_NC, _N, _NE), U, W, B_ne, B_en)

    x = _tc_readout(X, V0_w, V0_b, V1_w, V1_b)
    logits = _tc_logits(He, P3)
    He_T = _tc_het(He)
    return (x, He_T, logits)
